# Initial kernel scaffold; baseline (speedup 1.0000x reference)
import jax
import jax.numpy as jnp
from jax.experimental import pallas as pl


def kernel(features, adjacencies, adjacencies_t, w, bias, w_t, bias_t):
    raise NotImplementedError("write your pallas kernel here")


# speedup vs baseline: 2.0300x; 2.0300x over previous
import functools

import jax
import jax.numpy as jnp
from jax.experimental import pallas as pl
from jax.experimental.pallas import tpu as pltpu

N = 4096
D = 128
R = 2
BM = 512


def _body(feat_ref, adj_ref, adjt_ref, w_ref, b_ref, wt_ref, bt_ref,
          out_ref, g_scr, gt_scr, acc_scr, cnt_scr):
    m = pl.program_id(0)
    r = pl.program_id(1)

    @pl.when(m == 0)
    def _prologue():
        f = feat_ref[...]
        g_scr[r] = jnp.dot(f, w_ref[r], preferred_element_type=jnp.float32)
        gt_scr[r] = jnp.dot(f, wt_ref[r], preferred_element_type=jnp.float32)

    a = adj_ref[0]
    at = adjt_ref[0]
    af = a.astype(jnp.float32)
    atf = at.astype(jnp.float32)

    y = jnp.dot(af, g_scr[r], preferred_element_type=jnp.float32)
    yt = jnp.dot(atf, gt_scr[r], preferred_element_type=jnp.float32)

    deg = jnp.sum(af, axis=1, keepdims=True)
    degt = jnp.sum(atf, axis=1, keepdims=True)
    mask = (deg > 0.0).astype(jnp.float32)
    maskt = (degt > 0.0).astype(jnp.float32)

    bw = jnp.dot(b_ref[pl.ds(r, 1), :], w_ref[r],
                 preferred_element_type=jnp.float32)
    bwt = jnp.dot(bt_ref[pl.ds(r, 1), :], wt_ref[r],
                  preferred_element_type=jnp.float32)

    contrib = (y / jnp.maximum(deg, 1.0) + mask * bw
               + yt / jnp.maximum(degt, 1.0) + maskt * bwt)
    cnt = mask + maskt

    @pl.when(r == 0)
    def _init():
        acc_scr[...] = contrib
        cnt_scr[...] = cnt

    @pl.when(r == R - 1)
    def _finish():
        total = acc_scr[...] + contrib
        full = cnt_scr[...] + cnt
        out_ref[...] = total / jnp.where(full == 0.0, 1.0, full)


@jax.jit
def kernel(features, adjacencies, adjacencies_t, w, bias, w_t, bias_t):
    grid = (N // BM, R)
    return pl.pallas_call(
        _body,
        grid=grid,
        in_specs=[
            pl.BlockSpec((N, D), lambda m, r: (0, 0)),
            pl.BlockSpec((1, BM, N), lambda m, r: (r, m, 0)),
            pl.BlockSpec((1, BM, N), lambda m, r: (r, m, 0)),
            pl.BlockSpec((R, D, D), lambda m, r: (0, 0, 0)),
            pl.BlockSpec((R, D), lambda m, r: (0, 0)),
            pl.BlockSpec((R, D, D), lambda m, r: (0, 0, 0)),
            pl.BlockSpec((R, D), lambda m, r: (0, 0)),
        ],
        out_specs=pl.BlockSpec((BM, D), lambda m, r: (m, 0)),
        out_shape=jax.ShapeDtypeStruct((N, D), jnp.float32),
        scratch_shapes=[
            pltpu.VMEM((R, N, D), jnp.float32),
            pltpu.VMEM((R, N, D), jnp.float32),
            pltpu.VMEM((BM, D), jnp.float32),
            pltpu.VMEM((BM, 1), jnp.float32),
        ],
    )(features, adjacencies, adjacencies_t, w, bias, w_t, bias_t)


# speedup vs baseline: 2.0366x; 1.0032x over previous
import functools

import jax
import jax.numpy as jnp
from jax.experimental import pallas as pl
from jax.experimental.pallas import tpu as pltpu

N = 4096
D = 128
R = 2
BM = 512


def _body(feat_ref, adj_ref, adjt_ref, w_ref, b_ref, wt_ref, bt_ref,
          out_ref, g_scr, gt_scr, acc_scr, cnt_scr):
    m = pl.program_id(0)
    r = pl.program_id(1)

    @pl.when(m == 0)
    def _prologue():
        f = feat_ref[...]
        g_scr[r] = jnp.dot(f, w_ref[r],
                           preferred_element_type=jnp.float32).astype(jnp.bfloat16)
        gt_scr[r] = jnp.dot(f, wt_ref[r],
                            preferred_element_type=jnp.float32).astype(jnp.bfloat16)

    a = adj_ref[0]
    at = adjt_ref[0]
    ab = a.astype(jnp.bfloat16)
    atb = at.astype(jnp.bfloat16)

    y = jnp.dot(ab, g_scr[r], preferred_element_type=jnp.float32)
    yt = jnp.dot(atb, gt_scr[r], preferred_element_type=jnp.float32)

    deg = jnp.sum(a, axis=1, keepdims=True).astype(jnp.float32)
    degt = jnp.sum(at, axis=1, keepdims=True).astype(jnp.float32)
    mask = (deg > 0.0).astype(jnp.float32)
    maskt = (degt > 0.0).astype(jnp.float32)

    bw = jnp.dot(b_ref[pl.ds(r, 1), :], w_ref[r],
                 preferred_element_type=jnp.float32)
    bwt = jnp.dot(bt_ref[pl.ds(r, 1), :], wt_ref[r],
                  preferred_element_type=jnp.float32)

    contrib = (y / jnp.maximum(deg, 1.0) + mask * bw
               + yt / jnp.maximum(degt, 1.0) + maskt * bwt)
    cnt = mask + maskt

    @pl.when(r == 0)
    def _init():
        acc_scr[...] = contrib
        cnt_scr[...] = cnt

    @pl.when(r == R - 1)
    def _finish():
        total = acc_scr[...] + contrib
        full = cnt_scr[...] + cnt
        out_ref[...] = total / jnp.where(full == 0.0, 1.0, full)


@jax.jit
def kernel(features, adjacencies, adjacencies_t, w, bias, w_t, bias_t):
    grid = (N // BM, R)
    return pl.pallas_call(
        _body,
        grid=grid,
        in_specs=[
            pl.BlockSpec((N, D), lambda m, r: (0, 0)),
            pl.BlockSpec((1, BM, N), lambda m, r: (r, m, 0)),
            pl.BlockSpec((1, BM, N), lambda m, r: (r, m, 0)),
            pl.BlockSpec((R, D, D), lambda m, r: (0, 0, 0)),
            pl.BlockSpec((R, D), lambda m, r: (0, 0)),
            pl.BlockSpec((R, D, D), lambda m, r: (0, 0, 0)),
            pl.BlockSpec((R, D), lambda m, r: (0, 0)),
        ],
        out_specs=pl.BlockSpec((BM, D), lambda m, r: (m, 0)),
        out_shape=jax.ShapeDtypeStruct((N, D), jnp.float32),
        scratch_shapes=[
            pltpu.VMEM((R, N, D), jnp.bfloat16),
            pltpu.VMEM((R, N, D), jnp.bfloat16),
            pltpu.VMEM((BM, D), jnp.float32),
            pltpu.VMEM((BM, 1), jnp.float32),
        ],
    )(features, adjacencies, adjacencies_t, w, bias, w_t, bias_t)


# speedup vs baseline: 2.0869x; 1.0247x over previous
import functools

import jax
import jax.numpy as jnp
from jax.experimental import pallas as pl
from jax.experimental.pallas import tpu as pltpu

N = 4096
D = 128
R = 2
BM = 256


def _body(feat_ref, adj_ref, adjt_ref, w_ref, b_ref, wt_ref, bt_ref,
          out_ref, g_scr, gt_scr, acc_scr, cnt_scr):
    m = pl.program_id(0)
    r = pl.program_id(1)

    @pl.when(m == 0)
    def _prologue():
        f = feat_ref[...]
        g_scr[r] = jnp.dot(f, w_ref[r],
                           preferred_element_type=jnp.float32).astype(jnp.bfloat16)
        gt_scr[r] = jnp.dot(f, wt_ref[r],
                            preferred_element_type=jnp.float32).astype(jnp.bfloat16)

    a = adj_ref[0]
    at = adjt_ref[0]
    ab = a.astype(jnp.bfloat16)
    atb = at.astype(jnp.bfloat16)

    y = jnp.dot(ab, g_scr[r], preferred_element_type=jnp.float32)
    yt = jnp.dot(atb, gt_scr[r], preferred_element_type=jnp.float32)

    deg = jnp.sum(a, axis=1, keepdims=True).astype(jnp.float32)
    degt = jnp.sum(at, axis=1, keepdims=True).astype(jnp.float32)
    mask = (deg > 0.0).astype(jnp.float32)
    maskt = (degt > 0.0).astype(jnp.float32)

    bw = jnp.dot(b_ref[pl.ds(r, 1), :], w_ref[r],
                 preferred_element_type=jnp.float32)
    bwt = jnp.dot(bt_ref[pl.ds(r, 1), :], wt_ref[r],
                  preferred_element_type=jnp.float32)

    contrib = (y / jnp.maximum(deg, 1.0) + mask * bw
               + yt / jnp.maximum(degt, 1.0) + maskt * bwt)
    cnt = mask + maskt

    @pl.when(r == 0)
    def _init():
        acc_scr[...] = contrib
        cnt_scr[...] = cnt

    @pl.when(r == R - 1)
    def _finish():
        total = acc_scr[...] + contrib
        full = cnt_scr[...] + cnt
        out_ref[...] = total / jnp.where(full == 0.0, 1.0, full)


@jax.jit
def kernel(features, adjacencies, adjacencies_t, w, bias, w_t, bias_t):
    grid = (N // BM, R)
    return pl.pallas_call(
        _body,
        grid=grid,
        in_specs=[
            pl.BlockSpec((N, D), lambda m, r: (0, 0)),
            pl.BlockSpec((1, BM, N), lambda m, r: (r, m, 0)),
            pl.BlockSpec((1, BM, N), lambda m, r: (r, m, 0)),
            pl.BlockSpec((R, D, D), lambda m, r: (0, 0, 0)),
            pl.BlockSpec((R, D), lambda m, r: (0, 0)),
            pl.BlockSpec((R, D, D), lambda m, r: (0, 0, 0)),
            pl.BlockSpec((R, D), lambda m, r: (0, 0)),
        ],
        out_specs=pl.BlockSpec((BM, D), lambda m, r: (m, 0)),
        out_shape=jax.ShapeDtypeStruct((N, D), jnp.float32),
        scratch_shapes=[
            pltpu.VMEM((R, N, D), jnp.bfloat16),
            pltpu.VMEM((R, N, D), jnp.bfloat16),
            pltpu.VMEM((BM, D), jnp.float32),
            pltpu.VMEM((BM, 1), jnp.float32),
        ],
    )(features, adjacencies, adjacencies_t, w, bias, w_t, bias_t)
